# Initial kernel scaffold; baseline (speedup 1.0000x reference)
import jax
import jax.numpy as jnp
from jax.experimental import pallas as pl


def kernel(state, option, action, linear1, linear2, linear3):
    raise NotImplementedError("write your pallas kernel here")


# speedup vs baseline: 2.4078x; 2.4078x over previous
import jax
import jax.numpy as jnp
from jax import lax
from jax.experimental import pallas as pl
from jax.experimental.pallas import tpu as pltpu

_B = 1024
_NI = 512
_NO = 64
_H = 128


def _v_body(l1_ref, l2_ref, l3_ref, v_ref):
    l1b = l1_ref[0]
    l2b = l2_ref[0]
    l3b = l3_ref[0]
    u = lax.dot_general(l2b, l3b, (((1,), (0,)), ((), ())),
                        preferred_element_type=jnp.float32)
    vcol = lax.dot_general(l1b, u, (((1,), (0,)), ((), ())),
                           preferred_element_type=jnp.float32)
    v_ref[...] = vcol[None]


def _precompute_v(linear1, linear2, linear3):
    v3 = pl.pallas_call(
        _v_body,
        grid=(_NO,),
        in_specs=[
            pl.BlockSpec((1, _NI, _H), lambda o: (o, 0, 0)),
            pl.BlockSpec((1, _H, _H), lambda o: (o, 0, 0)),
            pl.BlockSpec((1, _H, 1), lambda o: (o, 0, 0)),
        ],
        out_specs=pl.BlockSpec((1, _NI, 1), lambda o: (o, 0, 0)),
        out_shape=jax.ShapeDtypeStruct((_NO, _NI, 1), jnp.float32),
    )(linear1, linear2, linear3)
    return v3.reshape(_NO, _NI)


def _select_body(state_ref, opt_ref, v_ref, out_ref):
    scores = lax.dot_general(state_ref[...], v_ref[...],
                             (((1,), (1,)), ((), ())),
                             preferred_element_type=jnp.float32)
    onehot = (opt_ref[...] == lax.broadcasted_iota(jnp.int32, (1, _NO), 1))
    out_ref[...] = jnp.sum(jnp.where(onehot, scores, 0.0), axis=1,
                           keepdims=True)


def kernel(state, option, action, linear1, linear2, linear3):
    v = _precompute_v(linear1, linear2, linear3)
    opt = option.astype(jnp.int32).reshape(_B, 1)
    out = pl.pallas_call(
        _select_body,
        out_shape=jax.ShapeDtypeStruct((_B, 1), jnp.float32),
    )(state, opt, v)
    return out


# speedup vs baseline: 4.9603x; 2.0601x over previous
import jax
import jax.numpy as jnp
from jax import lax
from jax.experimental import pallas as pl
from jax.experimental.pallas import tpu as pltpu

_B = 1024
_NI = 512
_NO = 64
_H = 128


_OB = 16


def _v_body(l1_ref, l2_ref, l3_ref, v_ref):
    l1b = l1_ref[...]
    l2b = l2_ref[...]
    l3b = l3_ref[...]
    u = lax.dot_general(l2b, l3b, (((2,), (1,)), ((0,), (0,))),
                        preferred_element_type=jnp.float32)
    vcol = lax.dot_general(l1b, u, (((2,), (1,)), ((0,), (0,))),
                           preferred_element_type=jnp.float32)
    v_ref[...] = vcol


def _precompute_v(linear1, linear2, linear3):
    v3 = pl.pallas_call(
        _v_body,
        grid=(_NO // _OB,),
        in_specs=[
            pl.BlockSpec((_OB, _NI, _H), lambda o: (o, 0, 0)),
            pl.BlockSpec((_OB, _H, _H), lambda o: (o, 0, 0)),
            pl.BlockSpec((_OB, _H, 1), lambda o: (o, 0, 0)),
        ],
        out_specs=pl.BlockSpec((_OB, _NI, 1), lambda o: (o, 0, 0)),
        out_shape=jax.ShapeDtypeStruct((_NO, _NI, 1), jnp.float32),
    )(linear1, linear2, linear3)
    return v3.reshape(_NO, _NI)


def _select_body(state_ref, opt_ref, v_ref, out_ref):
    scores = lax.dot_general(state_ref[...], v_ref[...],
                             (((1,), (1,)), ((), ())),
                             preferred_element_type=jnp.float32)
    onehot = (opt_ref[...] == lax.broadcasted_iota(jnp.int32, (1, _NO), 1))
    out_ref[...] = jnp.sum(jnp.where(onehot, scores, 0.0), axis=1,
                           keepdims=True)


def kernel(state, option, action, linear1, linear2, linear3):
    v = _precompute_v(linear1, linear2, linear3)
    opt = option.astype(jnp.int32).reshape(_B, 1)
    out = pl.pallas_call(
        _select_body,
        out_shape=jax.ShapeDtypeStruct((_B, 1), jnp.float32),
    )(state, opt, v)
    return out


# speedup vs baseline: 5.0899x; 1.0261x over previous
import jax
import jax.numpy as jnp
from jax import lax
from jax.experimental import pallas as pl
from jax.experimental.pallas import tpu as pltpu

_B = 1024
_NI = 512
_NO = 64
_H = 128


_OB = 32


def _v_body(l1_ref, l2_ref, l3_ref, v_ref):
    l1b = l1_ref[...]
    l2b = l2_ref[...]
    l3b = l3_ref[...]
    u = lax.dot_general(l2b, l3b, (((2,), (1,)), ((0,), (0,))),
                        preferred_element_type=jnp.float32)
    vcol = lax.dot_general(l1b, u, (((2,), (1,)), ((0,), (0,))),
                           preferred_element_type=jnp.float32)
    v_ref[...] = vcol


def _precompute_v(linear1, linear2, linear3):
    v3 = pl.pallas_call(
        _v_body,
        grid=(_NO // _OB,),
        in_specs=[
            pl.BlockSpec((_OB, _NI, _H), lambda o: (o, 0, 0)),
            pl.BlockSpec((_OB, _H, _H), lambda o: (o, 0, 0)),
            pl.BlockSpec((_OB, _H, 1), lambda o: (o, 0, 0)),
        ],
        out_specs=pl.BlockSpec((_OB, _NI, 1), lambda o: (o, 0, 0)),
        out_shape=jax.ShapeDtypeStruct((_NO, _NI, 1), jnp.float32),
    )(linear1, linear2, linear3)
    return v3.reshape(_NO, _NI)


def _select_body(state_ref, opt_ref, v_ref, out_ref):
    scores = lax.dot_general(state_ref[...], v_ref[...],
                             (((1,), (1,)), ((), ())),
                             preferred_element_type=jnp.float32)
    onehot = (opt_ref[...] == lax.broadcasted_iota(jnp.int32, (1, _NO), 1))
    out_ref[...] = jnp.sum(jnp.where(onehot, scores, 0.0), axis=1,
                           keepdims=True)


def kernel(state, option, action, linear1, linear2, linear3):
    v = _precompute_v(linear1, linear2, linear3)
    opt = option.astype(jnp.int32).reshape(_B, 1)
    out = pl.pallas_call(
        _select_body,
        out_shape=jax.ShapeDtypeStruct((_B, 1), jnp.float32),
    )(state, opt, v)
    return out


# speedup vs baseline: 8.6148x; 1.6925x over previous
import jax
import jax.numpy as jnp
from jax import lax
from jax.experimental import pallas as pl
from jax.experimental.pallas import tpu as pltpu

_B = 1024
_NI = 512
_NO = 64
_H = 128

_OB = 16
_G = _NO // _OB


def _fused_body(l1_ref, l2_ref, l3_ref, state_ref, opt_ref, out_ref, v_s):
    o = pl.program_id(0)

    @pl.when(o < _G)
    def _build_v():
        l1b = l1_ref[...]
        l2b = l2_ref[...]
        l3b = l3_ref[...]
        u = lax.dot_general(l3b, l2b, (((1,), (2,)), ((0,), (0,))),
                            preferred_element_type=jnp.float32)
        vrow = lax.dot_general(u, l1b, (((2,), (2,)), ((0,), (0,))),
                               preferred_element_type=jnp.float32)
        v_s[pl.ds(o * _OB, _OB), :] = vrow.reshape(_OB, _NI)

    @pl.when(o == _G)
    def _select():
        scores = lax.dot_general(state_ref[...], v_s[...],
                                 (((1,), (1,)), ((), ())),
                                 preferred_element_type=jnp.float32)
        onehot = (opt_ref[...] == lax.broadcasted_iota(jnp.int32, (1, _NO), 1))
        out_ref[...] = jnp.sum(jnp.where(onehot, scores, 0.0), axis=1,
                               keepdims=True)


def kernel(state, option, action, linear1, linear2, linear3):
    opt = option.astype(jnp.int32).reshape(_B, 1)
    clamp = lambda o: (jnp.minimum(o, _G - 1), 0, 0)
    out = pl.pallas_call(
        _fused_body,
        grid=(_G + 1,),
        in_specs=[
            pl.BlockSpec((_OB, _NI, _H), clamp),
            pl.BlockSpec((_OB, _H, _H), clamp),
            pl.BlockSpec((_OB, _H, 1), clamp),
            pl.BlockSpec((_B, _NI), lambda o: (0, 0)),
            pl.BlockSpec((_B, 1), lambda o: (0, 0)),
        ],
        out_specs=pl.BlockSpec((_B, 1), lambda o: (0, 0)),
        out_shape=jax.ShapeDtypeStruct((_B, 1), jnp.float32),
        scratch_shapes=[pltpu.VMEM((_NO, _NI), jnp.float32)],
    )(linear1, linear2, linear3, state, opt)
    return out
